# baseline (speedup 1.0000x reference)
import functools

import jax
import jax.numpy as jnp
from jax import lax
from jax.experimental import pallas as pl
from jax.experimental.pallas import tpu as pltpu
from jax.experimental.pallas import tpu_sc as plsc

NBUF = 2
K = 4


def _emb_body(n_sc, ch, d, idx_hbm, table_hbm, out_hbm, idx_v, rows_v, *sems):
    gsem = sems[:NBUF]
    wsem = sems[NBUF:]
    nc = 2
    wid = lax.axis_index("s") * nc + lax.axis_index("c")
    sc_rows = K * ch
    base = wid * n_sc * sc_rows
    pltpu.sync_copy(idx_hbm.at[wid], idx_v)

    def gather(s, k, b):
        return pltpu.make_async_copy(
            table_hbm.at[idx_v.at[s * K + k]],
            rows_v.at[b].at[pl.ds(k * ch, ch)], gsem[b])

    def fire(s, b):
        for k in range(K):
            gather(s, k, b).start()

    def drain(s, b):
        for k in range(K):
            gather(s, k, b).wait()

    def write(s, b):
        return pltpu.make_async_copy(
            rows_v.at[b], out_hbm.at[pl.ds(base + s * sc_rows, sc_rows)],
            wsem[b])

    for b in range(NBUF):
        fire(b, b)

    n_rounds = n_sc // NBUF

    def steady(t, carry):
        s0 = t * NBUF
        for b in range(NBUF):
            s = s0 + b
            drain(s, b)
            write(s, b).start()
            write(s, b).wait()
            fire(s + NBUF, b)
        return carry

    lax.fori_loop(0, n_rounds - 1, steady, 0)

    s0 = (n_rounds - 1) * NBUF
    for b in range(NBUF):
        s = s0 + b
        drain(s, b)
        write(s, b).start()
        write(s, b).wait()


def kernel(x, table):
    b0, b1 = x.shape
    v, d = table.shape
    b = b0 * b1
    nw = 32
    ch = 128
    b_per_w = b // nw
    n_ch = b_per_w // ch
    n_sc = n_ch // K
    assert b_per_w * nw == b and n_ch * ch == b_per_w
    assert n_sc * K == n_ch and n_sc % NBUF == 0

    idx = x.reshape(nw, n_ch, ch).astype(jnp.int32)

    mesh = plsc.VectorSubcoreMesh(core_axis_name="c", subcore_axis_name="s")
    emb = functools.partial(
        pl.kernel,
        mesh=mesh,
        out_type=jax.ShapeDtypeStruct((b, d), jnp.float32),
        scratch_types=(
            [pltpu.VMEM((n_ch, ch), jnp.int32),
             pltpu.VMEM((NBUF, K * ch, d), jnp.float32)]
            + [pltpu.SemaphoreType.DMA] * (2 * NBUF)
        ),
        compiler_params=pltpu.CompilerParams(use_tc_tiling_on_sc=False),
    )(functools.partial(_emb_body, n_sc, ch, d))

    out = emb(idx, table)
    return out.reshape(b0, b1, d)
